# baseline (speedup 1.0000x reference)
import functools

import jax
import jax.numpy as jnp
from jax import lax
from jax.experimental import pallas as pl
from jax.experimental.pallas import tpu as pltpu
from jax.experimental.pallas import tpu_sc as plsc

N = 10000
E = 320000
R2 = 400
H = 128
HH = H // 2
HB = HH + 16

NC = 2
NS = 16
GL = 128
G2 = 158
GP = G2 // 2
GH = G2 // 2
G = 79
E_PAD = NS * G2 * GL

XS_ROWS = 512
AGG_ROWS = 10112
ZR_A = XS_ROWS // NS
ZR_B = AGG_ROWS // NS

_SLOPE = (1.0 / 8.0 + 1.0 / 3.0) / 2.0

_sc_mesh = plsc.VectorSubcoreMesh(core_axis_name="c", subcore_axis_name="s")


def _tc_a_body(emb_ref, wn_ref, h_ref, hw_ref):
    x = emb_ref[...]
    nrm = jnp.sqrt(jnp.sum(x * x, axis=1, keepdims=True))
    h = x / jnp.maximum(nrm, 1e-12)
    h_ref[...] = h
    hw = jnp.dot(h, wn_ref[...], preferred_element_type=jnp.float32)
    z = jnp.zeros((N, HB - HH), jnp.float32)
    hw_ref[0] = jnp.concatenate([hw[:, :HH], z], axis=1)
    hw_ref[1] = jnp.concatenate([hw[:, HH:], z], axis=1)


def _tc_a(emb, wn):
    return pl.pallas_call(
        _tc_a_body,
        out_shape=(jax.ShapeDtypeStruct((N, H), jnp.float32),
                   jax.ShapeDtypeStruct((NC, N, HB), jnp.float32)),
    )(emb, wn)


@functools.partial(
    pl.kernel,
    out_type=(jax.ShapeDtypeStruct((NC * XS_ROWS, H), jnp.float32),
              jax.ShapeDtypeStruct((NC * XS_ROWS, 16), jnp.float32)),
    mesh=_sc_mesh,
    compiler_params=pltpu.CompilerParams(use_tc_tiling_on_sc=False),
    scratch_types=[
        pltpu.VMEM((G, GL), jnp.int32),
        pltpu.VMEM((G, GL), jnp.int32),
        pltpu.VMEM((GL, H), jnp.float32),
        pltpu.VMEM((GL, H), jnp.float32),
        pltpu.VMEM((GL, 16), jnp.float32),
        pltpu.VMEM_SHARED((XS_ROWS, H), jnp.float32),
        pltpu.VMEM_SHARED((XS_ROWS, 16), jnp.float32),
        pltpu.SemaphoreType.DMA,
        pltpu.SemaphoreType.DMA,
    ],
)
def _sc_segsum(h_hbm, rte_hbm, rseg_hbm, zrow_hbm, z16_hbm, ones_hbm,
               xs_out, cnt_out, gidx, sidx, rows0, rows1, onesv, xs_sh, cnt_sh,
               sg0, sg1):
    c = lax.axis_index("c")
    s = lax.axis_index("s")
    wid = s * NC + c
    pltpu.sync_copy(rte_hbm.at[wid], gidx)
    pltpu.sync_copy(rseg_hbm.at[wid], sidx)
    pltpu.sync_copy(ones_hbm, onesv)
    pltpu.sync_copy(zrow_hbm, xs_sh.at[pl.ds(s * ZR_A, ZR_A)])
    pltpu.sync_copy(z16_hbm, cnt_sh.at[pl.ds(s * ZR_A, ZR_A)])
    plsc.subcore_barrier()

    def fire_g(g, rows, sg):
        pltpu.async_copy(h_hbm.at[gidx.at[g]], rows, sg)

    def wait_g(rows, sg):
        pltpu.make_async_copy(h_hbm.at[gidx.at[0]], rows, sg).wait()

    def scatter(g, rows):
        pltpu.sync_copy(rows, xs_sh.at[sidx.at[g]], add=True)
        pltpu.sync_copy(onesv, cnt_sh.at[sidx.at[g]], add=True)

    fire_g(0, rows0, sg0)

    def body(p, carry):
        g0 = 2 * p
        wait_g(rows0, sg0)
        fire_g(g0 + 1, rows1, sg1)
        scatter(g0, rows0)
        wait_g(rows1, sg1)
        fire_g(lax.rem(g0 + 2, G), rows0, sg0)
        scatter(g0 + 1, rows1)
        return carry

    lax.fori_loop(0, G // 2, body, 0)
    wait_g(rows0, sg0)
    scatter(G - 1, rows0)
    plsc.subcore_barrier()
    off = c * XS_ROWS + s * ZR_A
    pltpu.sync_copy(xs_sh.at[pl.ds(s * ZR_A, ZR_A)], xs_out.at[pl.ds(off, ZR_A)])
    pltpu.sync_copy(cnt_sh.at[pl.ds(s * ZR_A, ZR_A)], cnt_out.at[pl.ds(off, ZR_A)])


def _tc_b_body(xs_ref, cnt_ref, er_ref, wih_ref, whh_ref, bih_ref, bhh_ref,
               wn_ref, h0w_ref):
    f32 = jnp.float32
    sums = xs_ref[0:R2, :] + xs_ref[XS_ROWS:XS_ROWS + R2, :]
    cnt = cnt_ref[0:R2, 0:1] + cnt_ref[XS_ROWS:XS_ROWS + R2, 0:1]
    x_mean = sums / jnp.maximum(cnt, 1.0)
    er = er_ref[...]
    wih = wih_ref[...]
    whh = whh_ref[...]
    dims = (((1,), (1,)), ((), ()))
    gi = (lax.dot_general(er, wih[:, :H], dims, preferred_element_type=f32)
          + lax.dot_general(x_mean, wih[:, H:], dims, preferred_element_type=f32)
          + bih_ref[...])
    gh = lax.dot_general(er, whh, dims, preferred_element_type=f32) + bhh_ref[...]
    r = jax.nn.sigmoid(gi[:, :H] + gh[:, :H])
    z = jax.nn.sigmoid(gi[:, H:2 * H] + gh[:, H:2 * H])
    n = jnp.tanh(gi[:, 2 * H:] + r * gh[:, 2 * H:])
    h0 = (1.0 - z) * n + z * er
    nrm = jnp.sqrt(jnp.sum(h0 * h0, axis=1, keepdims=True))
    h0 = h0 / jnp.maximum(nrm, 1e-12)
    h0w = jnp.dot(h0, wn_ref[...], preferred_element_type=f32)
    h0w_ref[0] = jnp.concatenate(
        [h0w[:, :HH], jnp.ones((R2, HB - HH), f32)], axis=1)
    h0w_ref[1] = jnp.concatenate(
        [h0w[:, HH:], jnp.zeros((R2, HB - HH), f32)], axis=1)


def _tc_b(xs, cnt, er, wih, whh, bih, bhh, wn):
    return pl.pallas_call(
        _tc_b_body,
        out_shape=jax.ShapeDtypeStruct((NC, R2, HB), jnp.float32),
    )(xs, cnt, er, wih, whh, bih, bhh, wn)


@functools.partial(
    pl.kernel,
    out_type=jax.ShapeDtypeStruct((NC * AGG_ROWS, HB), jnp.float32),
    mesh=_sc_mesh,
    compiler_params=pltpu.CompilerParams(use_tc_tiling_on_sc=False),
    scratch_types=[
        pltpu.VMEM((GP, GL), jnp.int32),
        pltpu.VMEM((GP, GL), jnp.int32),
        pltpu.VMEM((GP, GL), jnp.int32),
        pltpu.VMEM((GL, HB), jnp.float32),
        pltpu.VMEM((GL, HB), jnp.float32),
        pltpu.VMEM_SHARED((AGG_ROWS, HB), jnp.float32),
        pltpu.SemaphoreType.DMA,
        pltpu.SemaphoreType.DMA,
    ],
)
def _sc_agg(hw_hbm, h0w_hbm, src_hbm, dst_hbm, typ_hbm, zrow_hbm,
            agg_out, sidx, didx, tidx, rowsa, rowsb, agg_sh, sema, semb):
    c = lax.axis_index("c")
    s = lax.axis_index("s")
    wid = c * NS + s
    pltpu.sync_copy(zrow_hbm, agg_sh.at[pl.ds(s * ZR_B, ZR_B)])
    plsc.subcore_barrier()

    def phase(ph, carry):
        pltpu.sync_copy(src_hbm.at[wid * 2 + ph], sidx)
        pltpu.sync_copy(dst_hbm.at[s * 2 + ph], didx)
        pltpu.sync_copy(typ_hbm.at[wid * 2 + ph], tidx)

        def body(g, carry2):
            cpa = pltpu.async_copy(hw_hbm.at[sidx.at[g]], rowsa, sema)
            cpb = pltpu.async_copy(h0w_hbm.at[tidx.at[g]], rowsb, semb)
            cpa.wait()
            cpb.wait()
            pltpu.sync_copy(rowsa, agg_sh.at[didx.at[g]], add=True)
            pltpu.sync_copy(rowsb, agg_sh.at[didx.at[g]], add=True)
            return carry2

        lax.fori_loop(0, GP, body, 0)
        return carry

    lax.fori_loop(0, 2, phase, 0)
    plsc.subcore_barrier()
    off = c * AGG_ROWS + s * ZR_B
    pltpu.sync_copy(agg_sh.at[pl.ds(s * ZR_B, ZR_B)], agg_out.at[pl.ds(off, ZR_B)])


def _tc_c_body(agg_ref, h_ref, lw_ref, ew_ref, tw_ref, tb_ref, out_ref):
    f32 = jnp.float32
    agg = jnp.concatenate([agg_ref[0, :, :HH], agg_ref[1, :, :HH]], axis=1)
    deg = agg_ref[0, :, HH:HH + 1]
    h = h_ref[...]
    inv = 1.0 / jnp.maximum(deg, 1.0)
    loop_msg = jnp.where(
        deg > 0.0,
        jnp.dot(h, lw_ref[...], preferred_element_type=f32),
        jnp.dot(h, ew_ref[...], preferred_element_type=f32))
    nr = agg * inv + loop_msg
    nr = jnp.where(nr >= 0.0, nr, nr * _SLOPE)
    nrm = jnp.sqrt(jnp.sum(nr * nr, axis=1, keepdims=True))
    cur = nr / jnp.maximum(nrm, 1e-12)
    tw = jax.nn.sigmoid(jnp.dot(h, tw_ref[...], preferred_element_type=f32)
                        + tb_ref[...])
    out_ref[...] = tw * cur + (1.0 - tw) * h


def _tc_c(agg, h, lw, ew, tw, tb):
    rowb = 1000
    return pl.pallas_call(
        _tc_c_body,
        grid=(N // rowb,),
        in_specs=[
            pl.BlockSpec((NC, rowb, HB), lambda i: (0, i, 0)),
            pl.BlockSpec((rowb, H), lambda i: (i, 0)),
            pl.BlockSpec((H, H), lambda i: (0, 0)),
            pl.BlockSpec((H, H), lambda i: (0, 0)),
            pl.BlockSpec((H, H), lambda i: (0, 0)),
            pl.BlockSpec((1, H), lambda i: (0, 0)),
        ],
        out_specs=pl.BlockSpec((rowb, H), lambda i: (i, 0)),
        out_shape=jax.ShapeDtypeStruct((N, H), jnp.float32),
    )(agg, h, lw, ew, tw, tb)


def _pad_edges(a, pad_value):
    pad = jnp.full((E_PAD - E,), pad_value, a.dtype)
    return jnp.concatenate([a, pad])


def kernel(edge_src, edge_dst, edge_type, r_to_e, r_seg, dynamic_emb, emb_rel,
           weight_neighbor, loop_weight, evolve_loop_weight, time_gate_weight,
           time_gate_bias, gru_w_ih, gru_w_hh, gru_b_ih, gru_b_hh):
    f32 = jnp.float32
    ngrp = NC * NS * G
    rte = _pad_edges(r_to_e, 0).reshape(ngrp, GL).T.reshape(NC * NS, G, GL)
    rsg = _pad_edges(r_seg, R2).reshape(ngrp, GL).T.reshape(NC * NS, G, GL)
    src = _pad_edges(edge_src, 0).reshape(NS * 2, GP, GL)
    dst = _pad_edges(edge_dst, N).reshape(NS * 2, GP, GL)
    typ = _pad_edges(edge_type, 0).reshape(NS * 2, GP, GL)
    src2 = jnp.concatenate([src[None], src[None] + N]).reshape(NC * NS * 2, GP, GL)
    typ2 = jnp.concatenate([typ[None], typ[None] + R2]).reshape(NC * NS * 2, GP, GL)

    za_row = jnp.zeros((ZR_A, H), f32)
    za_16 = jnp.zeros((ZR_A, 16), f32)
    zb_row = jnp.zeros((ZR_B, HB), f32)
    ones = jnp.ones((GL, 16), f32)

    h, hw = _tc_a(dynamic_emb, weight_neighbor)
    xs, cnt = _sc_segsum(h, rte, rsg, za_row, za_16, ones)
    h0w = _tc_b(xs, cnt, emb_rel, gru_w_ih, gru_w_hh,
                gru_b_ih.reshape(1, 3 * H), gru_b_hh.reshape(1, 3 * H),
                weight_neighbor)
    agg = _sc_agg(hw.reshape(NC * N, HB), h0w.reshape(NC * R2, HB),
                  src2, dst, typ2, zb_row)
    agg = agg.reshape(NC, AGG_ROWS, HB)
    return _tc_c(agg, h, loop_weight, evolve_loop_weight,
                 time_gate_weight, time_gate_bias.reshape(1, H))


# speedup vs baseline: 1.1352x; 1.1352x over previous
import functools

import jax
import jax.numpy as jnp
from jax import lax
from jax.experimental import pallas as pl
from jax.experimental.pallas import tpu as pltpu
from jax.experimental.pallas import tpu_sc as plsc

N = 10000
E = 320000
R2 = 400
H = 128
HH = H // 2

NC = 2
NS = 16
GL = 128
G2 = 158
GH = G2 // 2
G = 79
E_PAD = NS * G2 * GL

XS_ROWS = 512
AGG_ROWS = 10112
ZR_A = XS_ROWS // NS
ZR_B = AGG_ROWS // NS

_SLOPE = (1.0 / 8.0 + 1.0 / 3.0) / 2.0

_sc_mesh = plsc.VectorSubcoreMesh(core_axis_name="c", subcore_axis_name="s")


def _tc_a_body(emb_ref, wn_ref, h_ref, hw_ref):
    x = emb_ref[...]
    nrm = jnp.sqrt(jnp.sum(x * x, axis=1, keepdims=True))
    h = x / jnp.maximum(nrm, 1e-12)
    h_ref[...] = h
    hw = jnp.dot(h, wn_ref[...], preferred_element_type=jnp.float32)
    hw_ref[0] = hw[:, :HH]
    hw_ref[1] = hw[:, HH:]


def _tc_a(emb, wn):
    return pl.pallas_call(
        _tc_a_body,
        out_shape=(jax.ShapeDtypeStruct((N, H), jnp.float32),
                   jax.ShapeDtypeStruct((NC, N, HH), jnp.float32)),
    )(emb, wn)


@functools.partial(
    pl.kernel,
    out_type=(jax.ShapeDtypeStruct((NC * XS_ROWS, H), jnp.float32),
              jax.ShapeDtypeStruct((NC * XS_ROWS, 16), jnp.float32)),
    mesh=_sc_mesh,
    compiler_params=pltpu.CompilerParams(use_tc_tiling_on_sc=False),
    scratch_types=[
        pltpu.VMEM((G, GL), jnp.int32),
        pltpu.VMEM((G, GL), jnp.int32),
        pltpu.VMEM((GL, H), jnp.float32),
        pltpu.VMEM((GL, H), jnp.float32),
        pltpu.VMEM((GL, 16), jnp.float32),
        pltpu.VMEM_SHARED((XS_ROWS, H), jnp.float32),
        pltpu.VMEM_SHARED((XS_ROWS, 16), jnp.float32),
        pltpu.SemaphoreType.DMA,
        pltpu.SemaphoreType.DMA,
    ],
)
def _sc_segsum(h_hbm, rte_hbm, rseg_hbm, zrow_hbm, z16_hbm, ones_hbm,
               xs_out, cnt_out, gidx, sidx, rows0, rows1, onesv, xs_sh, cnt_sh,
               sg0, sg1):
    c = lax.axis_index("c")
    s = lax.axis_index("s")
    wid = s * NC + c
    pltpu.sync_copy(rte_hbm.at[wid], gidx)
    pltpu.sync_copy(rseg_hbm.at[wid], sidx)
    pltpu.sync_copy(ones_hbm, onesv)
    pltpu.sync_copy(zrow_hbm, xs_sh.at[pl.ds(s * ZR_A, ZR_A)])
    pltpu.sync_copy(z16_hbm, cnt_sh.at[pl.ds(s * ZR_A, ZR_A)])
    plsc.subcore_barrier()

    def fire_g(g, rows, sg):
        pltpu.async_copy(h_hbm.at[gidx.at[g]], rows, sg)

    def wait_g(rows, sg):
        pltpu.make_async_copy(h_hbm.at[gidx.at[0]], rows, sg).wait()

    def scatter(g, rows):
        pltpu.sync_copy(rows, xs_sh.at[sidx.at[g]], add=True)
        pltpu.sync_copy(onesv, cnt_sh.at[sidx.at[g]], add=True)

    fire_g(0, rows0, sg0)

    def body(p, carry):
        g0 = 2 * p
        wait_g(rows0, sg0)
        fire_g(g0 + 1, rows1, sg1)
        scatter(g0, rows0)
        wait_g(rows1, sg1)
        fire_g(lax.rem(g0 + 2, G), rows0, sg0)
        scatter(g0 + 1, rows1)
        return carry

    lax.fori_loop(0, G // 2, body, 0)
    wait_g(rows0, sg0)
    scatter(G - 1, rows0)
    plsc.subcore_barrier()
    off = c * XS_ROWS + s * ZR_A
    pltpu.sync_copy(xs_sh.at[pl.ds(s * ZR_A, ZR_A)], xs_out.at[pl.ds(off, ZR_A)])
    pltpu.sync_copy(cnt_sh.at[pl.ds(s * ZR_A, ZR_A)], cnt_out.at[pl.ds(off, ZR_A)])


def _tc_b_body(xs_ref, cnt_ref, er_ref, wih_ref, whh_ref, bih_ref, bhh_ref,
               wn_ref, h0w_ref):
    f32 = jnp.float32
    sums = xs_ref[0:R2, :] + xs_ref[XS_ROWS:XS_ROWS + R2, :]
    cnt = cnt_ref[0:R2, 0:1] + cnt_ref[XS_ROWS:XS_ROWS + R2, 0:1]
    x_mean = sums / jnp.maximum(cnt, 1.0)
    er = er_ref[...]
    wih = wih_ref[...]
    whh = whh_ref[...]
    dims = (((1,), (1,)), ((), ()))
    gi = (lax.dot_general(er, wih[:, :H], dims, preferred_element_type=f32)
          + lax.dot_general(x_mean, wih[:, H:], dims, preferred_element_type=f32)
          + bih_ref[...])
    gh = lax.dot_general(er, whh, dims, preferred_element_type=f32) + bhh_ref[...]
    r = jax.nn.sigmoid(gi[:, :H] + gh[:, :H])
    z = jax.nn.sigmoid(gi[:, H:2 * H] + gh[:, H:2 * H])
    n = jnp.tanh(gi[:, 2 * H:] + r * gh[:, 2 * H:])
    h0 = (1.0 - z) * n + z * er
    nrm = jnp.sqrt(jnp.sum(h0 * h0, axis=1, keepdims=True))
    h0 = h0 / jnp.maximum(nrm, 1e-12)
    h0w = jnp.dot(h0, wn_ref[...], preferred_element_type=f32)
    h0w_ref[0] = h0w[:, :HH]
    h0w_ref[1] = h0w[:, HH:]


def _tc_b(xs, cnt, er, wih, whh, bih, bhh, wn):
    return pl.pallas_call(
        _tc_b_body,
        out_shape=jax.ShapeDtypeStruct((NC, R2, HH), jnp.float32),
    )(xs, cnt, er, wih, whh, bih, bhh, wn)


@functools.partial(
    pl.kernel,
    out_type=(jax.ShapeDtypeStruct((NC * AGG_ROWS, HH), jnp.float32),
              jax.ShapeDtypeStruct((NC * AGG_ROWS, 16), jnp.float32)),
    mesh=_sc_mesh,
    compiler_params=pltpu.CompilerParams(use_tc_tiling_on_sc=False),
    scratch_types=[
        pltpu.VMEM((G2, GL), jnp.int32),
        pltpu.VMEM((G2, GL), jnp.int32),
        pltpu.VMEM((G2, GL), jnp.int32),
        pltpu.VMEM((GL, HH), jnp.float32),
        pltpu.VMEM((GL, HH), jnp.float32),
        pltpu.VMEM((GL, 16), jnp.float32),
        pltpu.VMEM_SHARED((AGG_ROWS, HH), jnp.float32),
        pltpu.VMEM_SHARED((AGG_ROWS, 16), jnp.float32),
        pltpu.SemaphoreType.DMA,
        pltpu.SemaphoreType.DMA,
    ],
)
def _sc_agg(hw_hbm, h0w_hbm, src_hbm, dst_hbm, typ_hbm, zrow_hbm, z16_hbm,
            ones_hbm, agg_out, deg_out, sidx, didx, tidx, rowsa, rowsb, onesv,
            agg_sh, deg_sh, sema, semb):
    c = lax.axis_index("c")
    s = lax.axis_index("s")
    wid = c * NS + s
    pltpu.sync_copy(src_hbm.at[wid], sidx)
    pltpu.sync_copy(dst_hbm.at[s], didx)
    pltpu.sync_copy(typ_hbm.at[wid], tidx)
    pltpu.sync_copy(ones_hbm, onesv)
    pltpu.sync_copy(zrow_hbm, agg_sh.at[pl.ds(s * ZR_B, ZR_B)])
    pltpu.sync_copy(z16_hbm, deg_sh.at[pl.ds(s * ZR_B, ZR_B)])
    plsc.subcore_barrier()

    def body(g, carry):
        cpa = pltpu.async_copy(hw_hbm.at[sidx.at[g]], rowsa, sema)
        cpb = pltpu.async_copy(h0w_hbm.at[tidx.at[g]], rowsb, semb)
        cpa.wait()
        cpb.wait()
        pltpu.sync_copy(rowsa, agg_sh.at[didx.at[g]], add=True)
        pltpu.sync_copy(rowsb, agg_sh.at[didx.at[g]], add=True)

        @pl.when((g >= c * GH) & (g < (c + 1) * GH))
        def _():
            pltpu.sync_copy(onesv, deg_sh.at[didx.at[g]], add=True)

        return carry

    lax.fori_loop(0, G2, body, 0)
    plsc.subcore_barrier()
    off = c * AGG_ROWS + s * ZR_B
    pltpu.sync_copy(agg_sh.at[pl.ds(s * ZR_B, ZR_B)], agg_out.at[pl.ds(off, ZR_B)])
    pltpu.sync_copy(deg_sh.at[pl.ds(s * ZR_B, ZR_B)], deg_out.at[pl.ds(off, ZR_B)])


def _tc_c_body(agg_ref, deg_ref, h_ref, lw_ref, ew_ref, tw_ref, tb_ref, out_ref):
    f32 = jnp.float32
    agg = jnp.concatenate([agg_ref[0], agg_ref[1]], axis=1)
    deg = deg_ref[0, :, 0:1] + deg_ref[1, :, 0:1]
    h = h_ref[...]
    inv = 1.0 / jnp.maximum(deg, 1.0)
    loop_msg = jnp.where(
        deg > 0.0,
        jnp.dot(h, lw_ref[...], preferred_element_type=f32),
        jnp.dot(h, ew_ref[...], preferred_element_type=f32))
    nr = agg * inv + loop_msg
    nr = jnp.where(nr >= 0.0, nr, nr * _SLOPE)
    nrm = jnp.sqrt(jnp.sum(nr * nr, axis=1, keepdims=True))
    cur = nr / jnp.maximum(nrm, 1e-12)
    tw = jax.nn.sigmoid(jnp.dot(h, tw_ref[...], preferred_element_type=f32)
                        + tb_ref[...])
    out_ref[...] = tw * cur + (1.0 - tw) * h


def _tc_c(agg, deg, h, lw, ew, tw, tb):
    rowb = 1000
    return pl.pallas_call(
        _tc_c_body,
        grid=(N // rowb,),
        in_specs=[
            pl.BlockSpec((NC, rowb, HH), lambda i: (0, i, 0)),
            pl.BlockSpec((NC, rowb, 16), lambda i: (0, i, 0)),
            pl.BlockSpec((rowb, H), lambda i: (i, 0)),
            pl.BlockSpec((H, H), lambda i: (0, 0)),
            pl.BlockSpec((H, H), lambda i: (0, 0)),
            pl.BlockSpec((H, H), lambda i: (0, 0)),
            pl.BlockSpec((1, H), lambda i: (0, 0)),
        ],
        out_specs=pl.BlockSpec((rowb, H), lambda i: (i, 0)),
        out_shape=jax.ShapeDtypeStruct((N, H), jnp.float32),
    )(agg, deg, h, lw, ew, tw, tb)


def _pad_edges(a, pad_value):
    pad = jnp.full((E_PAD - E,), pad_value, a.dtype)
    return jnp.concatenate([a, pad])


def kernel(edge_src, edge_dst, edge_type, r_to_e, r_seg, dynamic_emb, emb_rel,
           weight_neighbor, loop_weight, evolve_loop_weight, time_gate_weight,
           time_gate_bias, gru_w_ih, gru_w_hh, gru_b_ih, gru_b_hh):
    f32 = jnp.float32
    ngrp = NC * NS * G
    rte = _pad_edges(r_to_e, 0).reshape(ngrp, GL).T.reshape(NC * NS, G, GL)
    rsg = _pad_edges(r_seg, R2).reshape(ngrp, GL).T.reshape(NC * NS, G, GL)
    src = _pad_edges(edge_src, 0).reshape(NS, G2, GL)
    dst = _pad_edges(edge_dst, N).reshape(NS, G2, GL)
    typ = _pad_edges(edge_type, 0).reshape(NS, G2, GL)
    src2 = jnp.concatenate([src[None], src[None] + N]).reshape(NC * NS, G2, GL)
    typ2 = jnp.concatenate([typ[None], typ[None] + R2]).reshape(NC * NS, G2, GL)

    za_row = jnp.zeros((ZR_A, H), f32)
    za_16 = jnp.zeros((ZR_A, 16), f32)
    zb_row = jnp.zeros((ZR_B, HH), f32)
    zb_16 = jnp.zeros((ZR_B, 16), f32)
    ones = jnp.ones((GL, 16), f32)

    h, hw = _tc_a(dynamic_emb, weight_neighbor)
    xs, cnt = _sc_segsum(h, rte, rsg, za_row, za_16, ones)
    h0w = _tc_b(xs, cnt, emb_rel, gru_w_ih, gru_w_hh,
                gru_b_ih.reshape(1, 3 * H), gru_b_hh.reshape(1, 3 * H),
                weight_neighbor)
    agg, deg = _sc_agg(hw.reshape(NC * N, HH), h0w.reshape(NC * R2, HH),
                       src2, dst, typ2, zb_row, zb_16, ones)
    agg = agg.reshape(NC, AGG_ROWS, HH)
    deg = deg.reshape(NC, AGG_ROWS, 16)
    return _tc_c(agg, deg, h, loop_weight, evolve_loop_weight,
                 time_gate_weight, time_gate_bias.reshape(1, H))


# speedup vs baseline: 1.1550x; 1.0175x over previous
import functools

import jax
import jax.numpy as jnp
from jax import lax
from jax.experimental import pallas as pl
from jax.experimental.pallas import tpu as pltpu
from jax.experimental.pallas import tpu_sc as plsc

N = 10000
E = 320000
R2 = 400
H = 128
HH = H // 2

NC = 2
NS = 16
GL = 128
G2 = 158
GH = G2 // 2
G = 79
E_PAD = NS * G2 * GL

XS_ROWS = 512
AGG_ROWS = 10112
ZR_A = XS_ROWS // NS
ZR_B = AGG_ROWS // NS

_SLOPE = (1.0 / 8.0 + 1.0 / 3.0) / 2.0

_sc_mesh = plsc.VectorSubcoreMesh(core_axis_name="c", subcore_axis_name="s")


def _tc_a_body(emb_ref, wn_ref, h_ref, hw_ref):
    x = emb_ref[...]
    nrm = jnp.sqrt(jnp.sum(x * x, axis=1, keepdims=True))
    h = x / jnp.maximum(nrm, 1e-12)
    h_ref[...] = h
    hw = jnp.dot(h, wn_ref[...], preferred_element_type=jnp.float32)
    hw_ref[0] = hw[:, :HH]
    hw_ref[1] = hw[:, HH:]


def _tc_a(emb, wn):
    return pl.pallas_call(
        _tc_a_body,
        out_shape=(jax.ShapeDtypeStruct((N, H), jnp.float32),
                   jax.ShapeDtypeStruct((NC, N, HH), jnp.float32)),
    )(emb, wn)


@functools.partial(
    pl.kernel,
    out_type=(jax.ShapeDtypeStruct((NC * XS_ROWS, H), jnp.float32),
              jax.ShapeDtypeStruct((NC * XS_ROWS, 16), jnp.float32)),
    mesh=_sc_mesh,
    compiler_params=pltpu.CompilerParams(use_tc_tiling_on_sc=False),
    scratch_types=[
        pltpu.VMEM((G, GL), jnp.int32),
        pltpu.VMEM((G, GL), jnp.int32),
        pltpu.VMEM((GL, H), jnp.float32),
        pltpu.VMEM((GL, H), jnp.float32),
        pltpu.VMEM((GL, 16), jnp.float32),
        pltpu.VMEM_SHARED((XS_ROWS, H), jnp.float32),
        pltpu.VMEM_SHARED((XS_ROWS, 16), jnp.float32),
        pltpu.SemaphoreType.DMA,
        pltpu.SemaphoreType.DMA,
    ],
)
def _sc_segsum(h_hbm, rte_hbm, rseg_hbm, zrow_hbm, z16_hbm, ones_hbm,
               xs_out, cnt_out, gidx, sidx, rows0, rows1, onesv, xs_sh, cnt_sh,
               sg0, sg1):
    c = lax.axis_index("c")
    s = lax.axis_index("s")
    wid = s * NC + c
    pltpu.sync_copy(rte_hbm.at[wid], gidx)
    pltpu.sync_copy(rseg_hbm.at[wid], sidx)
    pltpu.sync_copy(ones_hbm, onesv)
    pltpu.sync_copy(zrow_hbm, xs_sh.at[pl.ds(s * ZR_A, ZR_A)])
    pltpu.sync_copy(z16_hbm, cnt_sh.at[pl.ds(s * ZR_A, ZR_A)])
    plsc.subcore_barrier()

    def fire_g(g, rows, sg):
        pltpu.async_copy(h_hbm.at[gidx.at[g]], rows, sg)

    def wait_g(rows, sg):
        pltpu.make_async_copy(h_hbm.at[gidx.at[0]], rows, sg).wait()

    def scatter(g, rows):
        pltpu.sync_copy(rows, xs_sh.at[sidx.at[g]], add=True)
        pltpu.sync_copy(onesv, cnt_sh.at[sidx.at[g]], add=True)

    fire_g(0, rows0, sg0)

    def body(p, carry):
        g0 = 2 * p
        wait_g(rows0, sg0)
        fire_g(g0 + 1, rows1, sg1)
        scatter(g0, rows0)
        wait_g(rows1, sg1)
        fire_g(lax.rem(g0 + 2, G), rows0, sg0)
        scatter(g0 + 1, rows1)
        return carry

    lax.fori_loop(0, G // 2, body, 0)
    wait_g(rows0, sg0)
    scatter(G - 1, rows0)
    plsc.subcore_barrier()
    off = c * XS_ROWS + s * ZR_A
    pltpu.sync_copy(xs_sh.at[pl.ds(s * ZR_A, ZR_A)], xs_out.at[pl.ds(off, ZR_A)])
    pltpu.sync_copy(cnt_sh.at[pl.ds(s * ZR_A, ZR_A)], cnt_out.at[pl.ds(off, ZR_A)])


def _tc_b_body(xs_ref, cnt_ref, er_ref, wih_ref, whh_ref, bih_ref, bhh_ref,
               wn_ref, h0w_ref):
    f32 = jnp.float32
    sums = xs_ref[0:R2, :] + xs_ref[XS_ROWS:XS_ROWS + R2, :]
    cnt = cnt_ref[0:R2, 0:1] + cnt_ref[XS_ROWS:XS_ROWS + R2, 0:1]
    x_mean = sums / jnp.maximum(cnt, 1.0)
    er = er_ref[...]
    wih = wih_ref[...]
    whh = whh_ref[...]
    dims = (((1,), (1,)), ((), ()))
    gi = (lax.dot_general(er, wih[:, :H], dims, preferred_element_type=f32)
          + lax.dot_general(x_mean, wih[:, H:], dims, preferred_element_type=f32)
          + bih_ref[...])
    gh = lax.dot_general(er, whh, dims, preferred_element_type=f32) + bhh_ref[...]
    r = jax.nn.sigmoid(gi[:, :H] + gh[:, :H])
    z = jax.nn.sigmoid(gi[:, H:2 * H] + gh[:, H:2 * H])
    n = jnp.tanh(gi[:, 2 * H:] + r * gh[:, 2 * H:])
    h0 = (1.0 - z) * n + z * er
    nrm = jnp.sqrt(jnp.sum(h0 * h0, axis=1, keepdims=True))
    h0 = h0 / jnp.maximum(nrm, 1e-12)
    h0w = jnp.dot(h0, wn_ref[...], preferred_element_type=f32)
    h0w_ref[0] = h0w[:, :HH]
    h0w_ref[1] = h0w[:, HH:]


def _tc_b(xs, cnt, er, wih, whh, bih, bhh, wn):
    return pl.pallas_call(
        _tc_b_body,
        out_shape=jax.ShapeDtypeStruct((NC, R2, HH), jnp.float32),
    )(xs, cnt, er, wih, whh, bih, bhh, wn)


@functools.partial(
    pl.kernel,
    out_type=(jax.ShapeDtypeStruct((NC * AGG_ROWS, HH), jnp.float32),
              jax.ShapeDtypeStruct((NC * AGG_ROWS, 16), jnp.float32)),
    mesh=_sc_mesh,
    compiler_params=pltpu.CompilerParams(use_tc_tiling_on_sc=False),
    scratch_types=[
        pltpu.VMEM((G2, GL), jnp.int32),
        pltpu.VMEM((G2, GL), jnp.int32),
        pltpu.VMEM((G2, GL), jnp.int32),
        pltpu.VMEM((GL, HH), jnp.float32),
        pltpu.VMEM((GL, HH), jnp.float32),
        pltpu.VMEM((GL, 16), jnp.float32),
        pltpu.VMEM_SHARED((AGG_ROWS, HH), jnp.float32),
        pltpu.VMEM_SHARED((AGG_ROWS, 16), jnp.float32),
        pltpu.SemaphoreType.DMA,
        pltpu.SemaphoreType.DMA,
    ],
)
def _sc_agg(hw_hbm, h0w_hbm, src_hbm, dst_hbm, typ_hbm, zrow_hbm, z16_hbm,
            ones_hbm, agg_out, deg_out, sidx, didx, tidx, rowsa, rowsb, onesv,
            agg_sh, deg_sh, sema, semb):
    c = lax.axis_index("c")
    s = lax.axis_index("s")
    wid = c * NS + s
    pltpu.sync_copy(src_hbm.at[wid], sidx)
    pltpu.sync_copy(dst_hbm.at[s], didx)
    pltpu.sync_copy(typ_hbm.at[wid], tidx)
    pltpu.sync_copy(ones_hbm, onesv)
    pltpu.sync_copy(zrow_hbm, agg_sh.at[pl.ds(s * ZR_B, ZR_B)])
    pltpu.sync_copy(z16_hbm, deg_sh.at[pl.ds(s * ZR_B, ZR_B)])
    plsc.subcore_barrier()

    def body(g, carry):
        cpa = pltpu.async_copy(hw_hbm.at[sidx.at[g]], rowsa, sema)
        cpb = pltpu.async_copy(h0w_hbm.at[tidx.at[g]], rowsb, semb)

        @pl.when((g >= c * GH) & (g < (c + 1) * GH))
        def _():
            pltpu.sync_copy(onesv, deg_sh.at[didx.at[g]], add=True)

        cpa.wait()
        cpb.wait()
        pltpu.sync_copy(rowsa, agg_sh.at[didx.at[g]], add=True)
        pltpu.sync_copy(rowsb, agg_sh.at[didx.at[g]], add=True)
        return carry

    lax.fori_loop(0, G2, body, 0)
    plsc.subcore_barrier()
    off = c * AGG_ROWS + s * ZR_B
    pltpu.sync_copy(agg_sh.at[pl.ds(s * ZR_B, ZR_B)], agg_out.at[pl.ds(off, ZR_B)])
    pltpu.sync_copy(deg_sh.at[pl.ds(s * ZR_B, ZR_B)], deg_out.at[pl.ds(off, ZR_B)])


def _tc_c_body(agg_ref, deg_ref, h_ref, lw_ref, ew_ref, tw_ref, tb_ref, out_ref):
    f32 = jnp.float32
    agg = jnp.concatenate([agg_ref[0], agg_ref[1]], axis=1)
    deg = deg_ref[0, :, 0:1] + deg_ref[1, :, 0:1]
    h = h_ref[...]
    inv = 1.0 / jnp.maximum(deg, 1.0)
    loop_msg = jnp.where(
        deg > 0.0,
        jnp.dot(h, lw_ref[...], preferred_element_type=f32),
        jnp.dot(h, ew_ref[...], preferred_element_type=f32))
    nr = agg * inv + loop_msg
    nr = jnp.where(nr >= 0.0, nr, nr * _SLOPE)
    nrm = jnp.sqrt(jnp.sum(nr * nr, axis=1, keepdims=True))
    cur = nr / jnp.maximum(nrm, 1e-12)
    tw = jax.nn.sigmoid(jnp.dot(h, tw_ref[...], preferred_element_type=f32)
                        + tb_ref[...])
    out_ref[...] = tw * cur + (1.0 - tw) * h


def _tc_c(agg, deg, h, lw, ew, tw, tb):
    rowb = 1000
    return pl.pallas_call(
        _tc_c_body,
        grid=(N // rowb,),
        in_specs=[
            pl.BlockSpec((NC, rowb, HH), lambda i: (0, i, 0)),
            pl.BlockSpec((NC, rowb, 16), lambda i: (0, i, 0)),
            pl.BlockSpec((rowb, H), lambda i: (i, 0)),
            pl.BlockSpec((H, H), lambda i: (0, 0)),
            pl.BlockSpec((H, H), lambda i: (0, 0)),
            pl.BlockSpec((H, H), lambda i: (0, 0)),
            pl.BlockSpec((1, H), lambda i: (0, 0)),
        ],
        out_specs=pl.BlockSpec((rowb, H), lambda i: (i, 0)),
        out_shape=jax.ShapeDtypeStruct((N, H), jnp.float32),
    )(agg, deg, h, lw, ew, tw, tb)


def _pad_edges(a, pad_value):
    pad = jnp.full((E_PAD - E,), pad_value, a.dtype)
    return jnp.concatenate([a, pad])


def kernel(edge_src, edge_dst, edge_type, r_to_e, r_seg, dynamic_emb, emb_rel,
           weight_neighbor, loop_weight, evolve_loop_weight, time_gate_weight,
           time_gate_bias, gru_w_ih, gru_w_hh, gru_b_ih, gru_b_hh):
    f32 = jnp.float32
    ngrp = NC * NS * G
    rte = _pad_edges(r_to_e, 0).reshape(ngrp, GL).T.reshape(NC * NS, G, GL)
    rsg = _pad_edges(r_seg, R2).reshape(ngrp, GL).T.reshape(NC * NS, G, GL)
    src = _pad_edges(edge_src, 0).reshape(NS, G2, GL)
    dst = _pad_edges(edge_dst, N).reshape(NS, G2, GL)
    typ = _pad_edges(edge_type, 0).reshape(NS, G2, GL)
    src2 = jnp.concatenate([src[None], src[None] + N]).reshape(NC * NS, G2, GL)
    typ2 = jnp.concatenate([typ[None], typ[None] + R2]).reshape(NC * NS, G2, GL)

    za_row = jnp.zeros((ZR_A, H), f32)
    za_16 = jnp.zeros((ZR_A, 16), f32)
    zb_row = jnp.zeros((ZR_B, HH), f32)
    zb_16 = jnp.zeros((ZR_B, 16), f32)
    ones = jnp.ones((GL, 16), f32)

    h, hw = _tc_a(dynamic_emb, weight_neighbor)
    xs, cnt = _sc_segsum(h, rte, rsg, za_row, za_16, ones)
    h0w = _tc_b(xs, cnt, emb_rel, gru_w_ih, gru_w_hh,
                gru_b_ih.reshape(1, 3 * H), gru_b_hh.reshape(1, 3 * H),
                weight_neighbor)
    agg, deg = _sc_agg(hw.reshape(NC * N, HH), h0w.reshape(NC * R2, HH),
                       src2, dst, typ2, zb_row, zb_16, ones)
    agg = agg.reshape(NC, AGG_ROWS, HH)
    deg = deg.reshape(NC, AGG_ROWS, 16)
    return _tc_c(agg, deg, h, loop_weight, evolve_loop_weight,
                 time_gate_weight, time_gate_bias.reshape(1, H))
